# baseline (speedup 1.0000x reference)
import functools

import jax
import jax.numpy as jnp
from jax import lax
from jax.experimental import pallas as pl
from jax.experimental.pallas import tpu as pltpu
from jax.experimental.pallas import tpu_sc as plsc

VOCAB = 32128
D_MODEL = 1024
BATCH = 4
SEQ = 2048
N_TOK = BATCH * SEQ
NC, NS = 2, 16
NW = NC * NS
ROWS_PER_W = N_TOK // NW
CHUNK = 32
NCHUNK = ROWS_PER_W // CHUNK
PAD = 9
LANES = 16

DECODER_START_TOKEN_ID = 0
PAD_TOKEN_ID = 0


def _emb_body(label_hbm, table_hbm, out_hbm, mask_hbm,
              lbl_v, ids_v, buf0, buf1, buf2, ones_v,
              sg0, sg1, sg2, sw0, sw1, sw2):
    wid = lax.axis_index("s") * NC + lax.axis_index("c")
    base = pl.multiple_of(wid * ROWS_PER_W, ROWS_PER_W)
    b = wid // (SEQ // ROWS_PER_W)
    t_base = pl.multiple_of((wid % (SEQ // ROWS_PER_W)) * ROWS_PER_W,
                            ROWS_PER_W)

    pltpu.sync_copy(label_hbm.at[pl.ds(base, ROWS_PER_W + PAD - 1)], lbl_v)

    lane = lax.iota(jnp.int32, LANES)
    ones16 = jnp.full((LANES,), 1.0, jnp.float32)

    def idbody(j, c):
        n_vec = base + j * LANES + lane
        is_t0 = (n_vec & (SEQ - 1)) == 0
        ids = lbl_v[pl.ds(PAD - 1 + j * LANES, LANES)]
        ids = jnp.where(ids == -100, PAD_TOKEN_ID, ids)
        ids = jnp.where(is_t0, DECODER_START_TOKEN_ID, ids)
        ids_v[j // (CHUNK // LANES), pl.ds((j % (CHUNK // LANES)) * LANES, LANES)] = ids
        ones_v[pl.ds(j * LANES, LANES)] = ones16
        return c

    lax.fori_loop(0, ROWS_PER_W // LANES, idbody, 0)
    pltpu.sync_copy(ones_v, mask_hbm.at[b, pl.ds(t_base, ROWS_PER_W)])

    bufs, sgs, sws = [buf0, buf1, buf2], [sg0, sg1, sg2], [sw0, sw1, sw2]
    NBUF = 3

    def start_gather(k):
        return pltpu.async_copy(
            table_hbm.at[ids_v.at[k]], bufs[k % NBUF], sgs[k % NBUF])

    writes = [None] * NCHUNK
    gathers = [None] * NCHUNK
    gathers[0] = start_gather(0)
    gathers[1] = start_gather(1)
    for k in range(NCHUNK):
        gathers[k].wait()
        writes[k] = pltpu.async_copy(
            bufs[k % NBUF], out_hbm.at[b, pl.ds(t_base + k * CHUNK, CHUNK)],
            sws[k % NBUF])
        if k + 2 < NCHUNK:
            if k >= 1:
                writes[k - 1].wait()
            gathers[k + 2] = start_gather(k + 2)
    writes[NCHUNK - 3].wait()
    writes[NCHUNK - 2].wait()
    writes[NCHUNK - 1].wait()


@functools.partial(
    pl.kernel,
    out_type=(jax.ShapeDtypeStruct((BATCH, SEQ, D_MODEL), jnp.float32),
              jax.ShapeDtypeStruct((BATCH, SEQ), jnp.float32)),
    mesh=plsc.VectorSubcoreMesh(core_axis_name="c", subcore_axis_name="s",
                                num_cores=NC, num_subcores=NS),
    scratch_types=[
        pltpu.VMEM((ROWS_PER_W + PAD - 1,), jnp.int32),
        pltpu.VMEM((NCHUNK, CHUNK), jnp.int32),
        pltpu.VMEM((CHUNK, D_MODEL), jnp.float32),
        pltpu.VMEM((CHUNK, D_MODEL), jnp.float32),
        pltpu.VMEM((CHUNK, D_MODEL), jnp.float32),
        pltpu.VMEM((ROWS_PER_W,), jnp.float32),
        pltpu.SemaphoreType.DMA,
        pltpu.SemaphoreType.DMA,
        pltpu.SemaphoreType.DMA,
        pltpu.SemaphoreType.DMA,
        pltpu.SemaphoreType.DMA,
        pltpu.SemaphoreType.DMA,
    ],
)
def _emb_lookup(label_hbm, table_hbm, out_hbm, mask_hbm,
                lbl_v, ids_v, buf0, buf1, buf2, ones_v,
                sg0, sg1, sg2, sw0, sw1, sw2):
    _emb_body(label_hbm, table_hbm, out_hbm, mask_hbm,
              lbl_v, ids_v, buf0, buf1, buf2, ones_v,
              sg0, sg1, sg2, sw0, sw1, sw2)


def _copy_body(src_ref, am_ref, dst_ref, am_out_ref):
    dst_ref[...] = src_ref[...]
    am_out_ref[...] = am_ref[...]


_EHS_BLOCK = 512


def _tc_passthrough(x, attn_mask):
    flat = x.reshape(N_TOK, D_MODEL)
    spec = pl.BlockSpec((_EHS_BLOCK, D_MODEL), lambda i: (i, 0))
    am_spec = pl.BlockSpec((BATCH, SEQ), lambda i: (0, 0))
    out, am = pl.pallas_call(
        _copy_body,
        out_shape=(jax.ShapeDtypeStruct((N_TOK, D_MODEL), jnp.float32),
                   jax.ShapeDtypeStruct((BATCH, SEQ), jnp.float32)),
        grid=(N_TOK // _EHS_BLOCK,),
        in_specs=[spec, am_spec],
        out_specs=(spec, am_spec),
    )(flat, attn_mask)
    return out.reshape(BATCH, SEQ, D_MODEL), am


def kernel(encoder_hidden_states, label, encoder_attention_mask, embedding_table):
    label_padded = jnp.concatenate(
        [jnp.zeros((PAD,), jnp.int32), label.reshape(N_TOK)])
    out, mask = _emb_lookup(label_padded, embedding_table)
    ehs, attn = _tc_passthrough(encoder_hidden_states, encoder_attention_mask)
    return (ehs, attn, out, mask)


# speedup vs baseline: 1.0121x; 1.0121x over previous
import functools

import jax
import jax.numpy as jnp
from jax import lax
from jax.experimental import pallas as pl
from jax.experimental.pallas import tpu as pltpu
from jax.experimental.pallas import tpu_sc as plsc

VOCAB = 32128
D_MODEL = 1024
BATCH = 4
SEQ = 2048
N_TOK = BATCH * SEQ
NC, NS = 2, 16
NW = NC * NS
ROWS_PER_W = N_TOK // NW
CHUNK = 32
NCHUNK = ROWS_PER_W // CHUNK
PAD = 9
LANES = 16

DECODER_START_TOKEN_ID = 0
PAD_TOKEN_ID = 0


def _emb_body(label_hbm, table_hbm, out_hbm, mask_hbm,
              lbl_v, ids_v, buf0, buf1, ones_v, sg0, sg1, sw0, sw1):
    wid = lax.axis_index("s") * NC + lax.axis_index("c")
    base = pl.multiple_of(wid * ROWS_PER_W, ROWS_PER_W)
    b = wid // (SEQ // ROWS_PER_W)
    t_base = pl.multiple_of((wid % (SEQ // ROWS_PER_W)) * ROWS_PER_W,
                            ROWS_PER_W)

    pltpu.sync_copy(label_hbm.at[pl.ds(base, ROWS_PER_W + PAD - 1)], lbl_v)

    lane = lax.iota(jnp.int32, LANES)
    ones16 = jnp.full((LANES,), 1.0, jnp.float32)

    def idbody(j, c):
        n_vec = base + j * LANES + lane
        is_t0 = (n_vec & (SEQ - 1)) == 0
        ids = lbl_v[pl.ds(PAD - 1 + j * LANES, LANES)]
        ids = jnp.where(ids == -100, PAD_TOKEN_ID, ids)
        ids = jnp.where(is_t0, DECODER_START_TOKEN_ID, ids)
        ids_v[j // (CHUNK // LANES), pl.ds((j % (CHUNK // LANES)) * LANES, LANES)] = ids
        ones_v[pl.ds(j * LANES, LANES)] = ones16
        return c

    lax.fori_loop(0, ROWS_PER_W // LANES, idbody, 0)
    pltpu.sync_copy(ones_v, mask_hbm.at[b, pl.ds(t_base, ROWS_PER_W)])

    def start_gather(k, buf, sg):
        return pltpu.async_copy(table_hbm.at[ids_v.at[k]], buf, sg)

    def wait_gather(k, buf, sg):
        pltpu.make_async_copy(table_hbm.at[ids_v.at[k]], buf, sg).wait()

    def start_write(k, buf, sw):
        pltpu.async_copy(
            buf, out_hbm.at[b, pl.ds(t_base + k * CHUNK, CHUNK)], sw)

    def wait_write(buf, sw):
        pltpu.make_async_copy(
            buf, out_hbm.at[b, pl.ds(t_base, CHUNK)], sw).wait()

    start_gather(0, buf0, sg0)

    def ring(i, c):
        kk = i * 2
        wait_gather(kk, buf0, sg0)
        start_write(kk, buf0, sw0)

        @pl.when(kk > 0)
        def _():
            wait_write(buf1, sw1)
        start_gather(kk + 1, buf1, sg1)

        wait_gather(kk + 1, buf1, sg1)
        start_write(kk + 1, buf1, sw1)

        @pl.when(kk < NCHUNK - 2)
        def _():
            wait_write(buf0, sw0)
            start_gather(kk + 2, buf0, sg0)
        return c

    lax.fori_loop(0, NCHUNK // 2, ring, 0)
    wait_write(buf0, sw0)
    wait_write(buf1, sw1)


@functools.partial(
    pl.kernel,
    out_type=(jax.ShapeDtypeStruct((BATCH, SEQ, D_MODEL), jnp.float32),
              jax.ShapeDtypeStruct((BATCH, SEQ), jnp.float32)),
    mesh=plsc.VectorSubcoreMesh(core_axis_name="c", subcore_axis_name="s",
                                num_cores=NC, num_subcores=NS),
    scratch_types=[
        pltpu.VMEM((ROWS_PER_W + PAD - 1,), jnp.int32),
        pltpu.VMEM((NCHUNK, CHUNK), jnp.int32),
        pltpu.VMEM((CHUNK, D_MODEL), jnp.float32),
        pltpu.VMEM((CHUNK, D_MODEL), jnp.float32),
        pltpu.VMEM((ROWS_PER_W,), jnp.float32),
        pltpu.SemaphoreType.DMA,
        pltpu.SemaphoreType.DMA,
        pltpu.SemaphoreType.DMA,
        pltpu.SemaphoreType.DMA,
    ],
)
def _emb_lookup(label_hbm, table_hbm, out_hbm, mask_hbm,
                lbl_v, ids_v, buf0, buf1, ones_v, sg0, sg1, sw0, sw1):
    _emb_body(label_hbm, table_hbm, out_hbm, mask_hbm,
              lbl_v, ids_v, buf0, buf1, ones_v, sg0, sg1, sw0, sw1)


def _copy_body(src_ref, am_ref, dst_ref, am_out_ref):
    dst_ref[...] = src_ref[...]
    am_out_ref[...] = am_ref[...]


_EHS_BLOCK = 512


def _tc_passthrough(x, attn_mask):
    flat = x.reshape(N_TOK, D_MODEL)
    spec = pl.BlockSpec((_EHS_BLOCK, D_MODEL), lambda i: (i, 0))
    am_spec = pl.BlockSpec((BATCH, SEQ), lambda i: (0, 0))
    out, am = pl.pallas_call(
        _copy_body,
        out_shape=(jax.ShapeDtypeStruct((N_TOK, D_MODEL), jnp.float32),
                   jax.ShapeDtypeStruct((BATCH, SEQ), jnp.float32)),
        grid=(N_TOK // _EHS_BLOCK,),
        in_specs=[spec, am_spec],
        out_specs=(spec, am_spec),
    )(flat, attn_mask)
    return out.reshape(BATCH, SEQ, D_MODEL), am


def kernel(encoder_hidden_states, label, encoder_attention_mask, embedding_table):
    label_padded = jnp.concatenate(
        [jnp.zeros((PAD,), jnp.int32), label.reshape(N_TOK)])
    out, mask = _emb_lookup(label_padded, embedding_table)
    ehs, attn = _tc_passthrough(encoder_hidden_states, encoder_attention_mask)
    return (ehs, attn, out, mask)


# speedup vs baseline: 1.0122x; 1.0001x over previous
import functools

import jax
import jax.numpy as jnp
from jax import lax
from jax.experimental import pallas as pl
from jax.experimental.pallas import tpu as pltpu
from jax.experimental.pallas import tpu_sc as plsc

VOCAB = 32128
D_MODEL = 1024
BATCH = 4
SEQ = 2048
N_TOK = BATCH * SEQ
NC, NS = 2, 16
NW = NC * NS
ROWS_PER_W = N_TOK // NW
CHUNK = 32
NCHUNK = ROWS_PER_W // CHUNK
PAD = 9
LANES = 16

DECODER_START_TOKEN_ID = 0
PAD_TOKEN_ID = 0


def _emb_body(label_hbm, table_hbm, out_hbm, mask_hbm,
              lbl_v, ids_v, buf0, buf1, ones_v, sg0, sg1, sw0, sw1):
    wid = lax.axis_index("s") * NC + lax.axis_index("c")
    base = pl.multiple_of(wid * ROWS_PER_W, ROWS_PER_W)
    b = wid // (SEQ // ROWS_PER_W)
    t_base = pl.multiple_of((wid % (SEQ // ROWS_PER_W)) * ROWS_PER_W,
                            ROWS_PER_W)

    pltpu.sync_copy(label_hbm.at[pl.ds(base, ROWS_PER_W + PAD - 1)], lbl_v)

    lane = lax.iota(jnp.int32, LANES)
    ones16 = jnp.full((LANES,), 1.0, jnp.float32)

    def idbody(j, c):
        n_vec = base + j * LANES + lane
        is_t0 = (n_vec & (SEQ - 1)) == 0
        ids = lbl_v[pl.ds(PAD - 1 + j * LANES, LANES)]
        ids = jnp.where(ids == -100, PAD_TOKEN_ID, ids)
        ids = jnp.where(is_t0, DECODER_START_TOKEN_ID, ids)
        ids_v[j // (CHUNK // LANES), pl.ds((j % (CHUNK // LANES)) * LANES, LANES)] = ids
        ones_v[pl.ds(j * LANES, LANES)] = ones16
        return c

    lax.fori_loop(0, ROWS_PER_W // LANES, idbody, 0)
    pltpu.sync_copy(ones_v, mask_hbm.at[b, pl.ds(t_base, ROWS_PER_W)])

    def start_gather(k, buf, sg):
        return pltpu.async_copy(table_hbm.at[ids_v.at[k]], buf, sg)

    def wait_gather(k, buf, sg):
        pltpu.make_async_copy(table_hbm.at[ids_v.at[k]], buf, sg).wait()

    def start_write(k, buf, sw):
        pltpu.async_copy(
            buf, out_hbm.at[b, pl.ds(t_base + k * CHUNK, CHUNK)], sw)

    def wait_write(buf, sw):
        pltpu.make_async_copy(
            buf, out_hbm.at[b, pl.ds(t_base, CHUNK)], sw).wait()

    start_gather(0, buf0, sg0)

    def ring(i, c):
        kk = i * 2
        wait_gather(kk, buf0, sg0)
        start_write(kk, buf0, sw0)

        @pl.when(kk > 0)
        def _():
            wait_write(buf1, sw1)
        start_gather(kk + 1, buf1, sg1)

        wait_gather(kk + 1, buf1, sg1)
        start_write(kk + 1, buf1, sw1)

        @pl.when(kk < NCHUNK - 2)
        def _():
            wait_write(buf0, sw0)
            start_gather(kk + 2, buf0, sg0)
        return c

    lax.fori_loop(0, NCHUNK // 2, ring, 0)
    wait_write(buf0, sw0)
    wait_write(buf1, sw1)


@functools.partial(
    pl.kernel,
    out_type=(jax.ShapeDtypeStruct((BATCH, SEQ, D_MODEL), jnp.float32),
              jax.ShapeDtypeStruct((BATCH, SEQ), jnp.float32)),
    mesh=plsc.VectorSubcoreMesh(core_axis_name="c", subcore_axis_name="s",
                                num_cores=NC, num_subcores=NS),
    scratch_types=[
        pltpu.VMEM((ROWS_PER_W + PAD - 1,), jnp.int32),
        pltpu.VMEM((NCHUNK, CHUNK), jnp.int32),
        pltpu.VMEM((CHUNK, D_MODEL), jnp.float32),
        pltpu.VMEM((CHUNK, D_MODEL), jnp.float32),
        pltpu.VMEM((ROWS_PER_W,), jnp.float32),
        pltpu.SemaphoreType.DMA,
        pltpu.SemaphoreType.DMA,
        pltpu.SemaphoreType.DMA,
        pltpu.SemaphoreType.DMA,
    ],
)
def _emb_lookup(label_hbm, table_hbm, out_hbm, mask_hbm,
                lbl_v, ids_v, buf0, buf1, ones_v, sg0, sg1, sw0, sw1):
    _emb_body(label_hbm, table_hbm, out_hbm, mask_hbm,
              lbl_v, ids_v, buf0, buf1, ones_v, sg0, sg1, sw0, sw1)


def _copy_body(src_ref, am_ref, dst_ref, am_out_ref):
    dst_ref[...] = src_ref[...]
    am_out_ref[...] = am_ref[...]


_EHS_BLOCK = 512


def _tc_passthrough(x, attn_mask):
    flat = x.reshape(N_TOK, D_MODEL)
    spec = pl.BlockSpec((_EHS_BLOCK, D_MODEL), lambda i: (i, 0))
    am_spec = pl.BlockSpec((BATCH, SEQ), lambda i: (0, 0))
    out, am = pl.pallas_call(
        _copy_body,
        out_shape=(jax.ShapeDtypeStruct((N_TOK, D_MODEL), jnp.float32),
                   jax.ShapeDtypeStruct((BATCH, SEQ), jnp.float32)),
        grid=(N_TOK // _EHS_BLOCK,),
        in_specs=[spec, am_spec],
        out_specs=(spec, am_spec),
    )(flat, attn_mask)
    return out.reshape(BATCH, SEQ, D_MODEL), am


def kernel(encoder_hidden_states, label, encoder_attention_mask, embedding_table):
    label_padded = jnp.concatenate(
        [jnp.zeros((PAD,), jnp.int32), label.reshape(N_TOK).astype(jnp.int32)])
    out, mask = _emb_lookup(label_padded, embedding_table)
    ehs, attn = _tc_passthrough(encoder_hidden_states, encoder_attention_mask)
    return (ehs, attn, out, mask)
